# Initial kernel scaffold; baseline (speedup 1.0000x reference)
import jax
import jax.numpy as jnp
from jax.experimental import pallas as pl


def kernel(node_ids, edge_ids, node_table, edge_table):
    raise NotImplementedError("write your pallas kernel here")


# speedup vs baseline: 3.0284x; 3.0284x over previous
import functools

import jax
import jax.numpy as jnp
from jax import lax
from jax.experimental import pallas as pl
from jax.experimental.pallas import tpu as pltpu
from jax.experimental.pallas import tpu_sc as plsc

_N_NODES = 10000
_N_EDGES = 320000
_NODE_DIM = 128
_E_DIM = 16
_E_VOCAB = 16

_INFO = plsc.get_sparse_core_info()
_NC, _NS = _INFO.num_cores, _INFO.num_subcores
_NW = _NC * _NS
_L = 16

_NODE_CHUNK = 320
_EDGES_PER_W = _N_EDGES // _NW
_E_CHUNK = 2000
_N_E_CHUNKS = _EDGES_PER_W // _E_CHUNK
_GROUPS = _E_CHUNK // _L


def _sc_lookup(node_ids, edge_ids, node_table, edge_table_flat):
    mesh = plsc.VectorSubcoreMesh(core_axis_name="c", subcore_axis_name="s")

    @functools.partial(
        pl.kernel,
        mesh=mesh,
        out_type=(
            jax.ShapeDtypeStruct((_N_NODES, _NODE_DIM), jnp.float32),
            jax.ShapeDtypeStruct((_N_EDGES * _E_DIM,), jnp.float32),
        ),
        scratch_types=[
            pltpu.VMEM((_NODE_CHUNK,), jnp.int32),
            pltpu.VMEM((_NODE_CHUNK, _NODE_DIM), jnp.float32),
            pltpu.VMEM((_E_CHUNK,), jnp.int32),
            pltpu.VMEM((_E_CHUNK * _E_DIM,), jnp.float32),
            pltpu.VMEM((_E_VOCAB * _E_DIM,), jnp.float32),
            pltpu.SemaphoreType.DMA,
        ],
        compiler_params=pltpu.CompilerParams(needs_layout_passes=False),
    )
    def k(node_ids_hbm, edge_ids_hbm, node_tab_hbm, edge_tab_hbm,
          node_out, edge_out, nidx_v, nrows_v, eidx_v, erows_v, etab_v, sem):
        wid = lax.axis_index("s") * _NC + lax.axis_index("c")

        nbase = jnp.where(wid < _NW - 1,
                          wid * _NODE_CHUNK, _N_NODES - _NODE_CHUNK)
        pltpu.sync_copy(node_ids_hbm.at[pl.ds(nbase, _NODE_CHUNK)], nidx_v)
        pltpu.async_copy(node_tab_hbm.at[nidx_v], nrows_v, sem).wait()
        pltpu.sync_copy(nrows_v, node_out.at[pl.ds(nbase, _NODE_CHUNK)])

        pltpu.sync_copy(edge_tab_hbm, etab_v)
        lanes = lax.iota(jnp.int32, _L)

        def do_group(g, _):
            idvec = eidx_v[pl.ds(g * _L, _L)]
            src_base = idvec * _E_DIM
            dst_base = (g * _L + lanes) * _E_DIM
            for j in range(_E_DIM):
                vals = plsc.load_gather(etab_v, [src_base + j])
                plsc.store_scatter(erows_v, [dst_base + j], vals)
            return 0

        for c in range(_N_E_CHUNKS):
            ebase = wid * _EDGES_PER_W + c * _E_CHUNK
            pltpu.sync_copy(edge_ids_hbm.at[pl.ds(ebase, _E_CHUNK)], eidx_v)
            lax.fori_loop(0, _GROUPS, do_group, 0)
            pltpu.sync_copy(
                erows_v, edge_out.at[pl.ds(ebase * _E_DIM, _E_CHUNK * _E_DIM)])

    return k(node_ids, edge_ids, node_table, edge_table_flat)


def kernel(node_ids, edge_ids, node_table, edge_table):
    node_feat, edge_flat = _sc_lookup(
        node_ids, edge_ids, node_table, edge_table.reshape(-1))
    return node_feat, edge_flat.reshape(_N_EDGES, _E_DIM)


# speedup vs baseline: 3.1531x; 1.0412x over previous
import functools

import jax
import jax.numpy as jnp
from jax import lax
from jax.experimental import pallas as pl
from jax.experimental.pallas import tpu as pltpu
from jax.experimental.pallas import tpu_sc as plsc

_N_NODES = 10000
_N_EDGES = 320000
_NODE_DIM = 128
_E_DIM = 16
_E_VOCAB = 16

_INFO = plsc.get_sparse_core_info()
_NC, _NS = _INFO.num_cores, _INFO.num_subcores
_NW = _NC * _NS
_L = 16

_NODE_CHUNK = 320
_EDGES_PER_W = _N_EDGES // _NW
_E_CHUNK = 2000
_N_E_CHUNKS = _EDGES_PER_W // _E_CHUNK
_GROUPS = _E_CHUNK // _L
_UNROLL = 5


def _sc_lookup(node_ids, edge_ids, node_table, edge_table_flat):
    mesh = plsc.VectorSubcoreMesh(core_axis_name="c", subcore_axis_name="s")

    @functools.partial(
        pl.kernel,
        mesh=mesh,
        out_type=(
            jax.ShapeDtypeStruct((_N_NODES, _NODE_DIM), jnp.float32),
            jax.ShapeDtypeStruct((_N_EDGES * _E_DIM,), jnp.float32),
        ),
        scratch_types=[
            pltpu.VMEM((_NODE_CHUNK,), jnp.int32),
            pltpu.VMEM((_NODE_CHUNK, _NODE_DIM), jnp.float32),
            pltpu.VMEM((_E_CHUNK,), jnp.int32),
            pltpu.VMEM((_E_CHUNK,), jnp.int32),
            pltpu.VMEM((_E_CHUNK * _E_DIM,), jnp.float32),
            pltpu.VMEM((_E_CHUNK * _E_DIM,), jnp.float32),
            pltpu.VMEM((_E_VOCAB, _E_DIM), jnp.float32),
            pltpu.VMEM((_E_VOCAB * _E_DIM,), jnp.float32),
            pltpu.SemaphoreType.DMA,
            pltpu.SemaphoreType.DMA,
            pltpu.SemaphoreType.DMA,
            pltpu.SemaphoreType.DMA,
            pltpu.SemaphoreType.DMA,
            pltpu.SemaphoreType.DMA,
            pltpu.SemaphoreType.DMA,
        ],
        compiler_params=pltpu.CompilerParams(needs_layout_passes=False),
    )
    def k(node_ids_hbm, edge_ids_hbm, node_tab_hbm, edge_tab_hbm,
          node_out, edge_out, nidx_v, nrows_v, eidx0, eidx1, erow0, erow1,
          etab2_v, etab_v, sn_id, sn_g, sn_w, se_id0, se_id1, se_w0, se_w1):
        wid = lax.axis_index("s") * _NC + lax.axis_index("c")
        nbase = jnp.where(wid < _NW - 1,
                          wid * _NODE_CHUNK, _N_NODES - _NODE_CHUNK)
        ebase = wid * _EDGES_PER_W

        nid_cp = pltpu.async_copy(
            node_ids_hbm.at[pl.ds(nbase, _NODE_CHUNK)], nidx_v, sn_id)
        eidx = (eidx0, eidx1)
        erow = (erow0, erow1)
        seid = (se_id0, se_id1)
        sew = (se_w0, se_w1)
        ecp = [
            pltpu.async_copy(
                edge_ids_hbm.at[pl.ds(ebase + c * _E_CHUNK, _E_CHUNK)],
                eidx[c], seid[c])
            for c in range(2)
        ]
        pltpu.sync_copy(edge_tab_hbm, etab2_v)
        for r in range(_E_VOCAB):
            etab_v[pl.ds(r * _E_DIM, _E_DIM)] = etab2_v[r, :]
        nid_cp.wait()
        ng = pltpu.async_copy(node_tab_hbm.at[nidx_v], nrows_v, sn_g)

        lanes16 = lax.iota(jnp.int32, _L) * _E_DIM
        edge_out_flat = edge_out

        def expand_chunk(ids_ref, rows_ref):
            def body(it, _):
                for u in range(_UNROLL):
                    g = it * _UNROLL + u
                    idvec = ids_ref[pl.ds(g * _L, _L)]
                    src = idvec * _E_DIM
                    dst = g * (_L * _E_DIM) + lanes16
                    for j in range(_E_DIM):
                        vals = plsc.load_gather(etab_v, [src + j])
                        plsc.store_scatter(rows_ref, [dst + j], vals)
                return 0
            lax.fori_loop(0, _GROUPS // _UNROLL, body, 0)

        wr = [None, None]
        nw = None
        for c in range(_N_E_CHUNKS):
            b = c & 1
            ecp[b].wait()
            if wr[b] is not None:
                wr[b].wait()
            expand_chunk(eidx[b], erow[b])
            if c + 2 < _N_E_CHUNKS:
                ecp[b] = pltpu.async_copy(
                    edge_ids_hbm.at[pl.ds(ebase + (c + 2) * _E_CHUNK,
                                          _E_CHUNK)],
                    eidx[b], seid[b])
            wr[b] = pltpu.async_copy(
                erow[b],
                edge_out_flat.at[pl.ds((ebase + c * _E_CHUNK) * _E_DIM,
                                       _E_CHUNK * _E_DIM)],
                sew[b])
            if c == 0:
                ng.wait()
                nw = pltpu.async_copy(
                    nrows_v, node_out.at[pl.ds(nbase, _NODE_CHUNK)], sn_w)
        wr[0].wait()
        wr[1].wait()
        nw.wait()

    return k(node_ids, edge_ids, node_table, edge_table_flat)


def kernel(node_ids, edge_ids, node_table, edge_table):
    node_feat, edge_flat = _sc_lookup(node_ids, edge_ids, node_table,
                                      edge_table)
    return node_feat, edge_flat.reshape(_N_EDGES, _E_DIM)


# speedup vs baseline: 6.9566x; 2.2063x over previous
import functools

import jax
import jax.numpy as jnp
from jax import lax
from jax.experimental import pallas as pl
from jax.experimental.pallas import tpu as pltpu
from jax.experimental.pallas import tpu_sc as plsc

_N_NODES = 10000
_N_EDGES = 320000
_NODE_DIM = 128
_E_DIM = 16
_E_VOCAB = 16

_INFO = plsc.get_sparse_core_info()
_NC, _NS = _INFO.num_cores, _INFO.num_subcores
_NW = _NC * _NS
_L = 16

_NODE_CHUNK = 320

_BLK = 128
_N_BLKS = _N_EDGES // _BLK
_NB_W = 79
_TB = 8
_CHUNK_SIZES = (8, 8, 8, 8, 8, 8, 8, 8, 8, 7)
_CBUF_PLANE = _TB * _BLK * 8
_PLANE_STRIDE = _N_BLKS * _BLK * 8
_UNROLL = 4


def _sc_lookup(node_ids, edge_ids, node_table, edge_table):
    mesh = plsc.VectorSubcoreMesh(core_axis_name="c", subcore_axis_name="s")

    @functools.partial(
        pl.kernel,
        mesh=mesh,
        out_type=(
            jax.ShapeDtypeStruct((_N_NODES, _NODE_DIM), jnp.float32),
            jax.ShapeDtypeStruct((_N_EDGES * _E_DIM,), jnp.float32),
        ),
        scratch_types=[
            pltpu.VMEM((_NODE_CHUNK,), jnp.int32),
            pltpu.VMEM((_NODE_CHUNK, _NODE_DIM), jnp.float32),
            pltpu.VMEM((_TB * _BLK,), jnp.int32),
            pltpu.VMEM((_TB * _BLK,), jnp.int32),
            pltpu.VMEM((2 * _CBUF_PLANE,), jnp.float32),
            pltpu.VMEM((2 * _CBUF_PLANE,), jnp.float32),
            pltpu.VMEM((_E_VOCAB, _E_DIM), jnp.float32),
            pltpu.VMEM((_E_VOCAB * _E_DIM,), jnp.float32),
            pltpu.SemaphoreType.DMA,
            pltpu.SemaphoreType.DMA,
            pltpu.SemaphoreType.DMA,
            pltpu.SemaphoreType.DMA,
            pltpu.SemaphoreType.DMA,
            pltpu.SemaphoreType.DMA,
            pltpu.SemaphoreType.DMA,
        ],
        compiler_params=pltpu.CompilerParams(needs_layout_passes=False),
    )
    def k(node_ids_hbm, edge_ids_hbm, node_tab_hbm, edge_tab_hbm,
          node_out, edge_out, nidx_v, nrows_v, eidx0, eidx1, cbuf0, cbuf1,
          etab2_v, etab_v, sn_id, sn_g, sn_w, se_id0, se_id1, se_w0, se_w1):
        wid = lax.axis_index("s") * _NC + lax.axis_index("c")
        nbase = jnp.where(wid < _NW - 1,
                          wid * _NODE_CHUNK, _N_NODES - _NODE_CHUNK)
        bstart = (wid * _N_BLKS) // _NW

        nid_cp = pltpu.async_copy(
            node_ids_hbm.at[pl.ds(nbase, _NODE_CHUNK)], nidx_v, sn_id)
        eidx = (eidx0, eidx1)
        cbuf = (cbuf0, cbuf1)
        seid = (se_id0, se_id1)
        sew = (se_w0, se_w1)
        csum = [sum(_CHUNK_SIZES[:n]) for n in range(len(_CHUNK_SIZES))]
        ecp = [
            pltpu.async_copy(
                edge_ids_hbm.at[pl.ds((bstart + csum[c]) * _BLK,
                                      _CHUNK_SIZES[c] * _BLK)],
                eidx[c].at[pl.ds(0, _CHUNK_SIZES[c] * _BLK)], seid[c])
            for c in range(2)
        ]
        pltpu.sync_copy(edge_tab_hbm, etab2_v)
        for r in range(_E_VOCAB):
            etab_v[pl.ds(r * _E_DIM, _E_DIM)] = etab2_v[r, :]
        nid_cp.wait()
        ng = pltpu.async_copy(node_tab_hbm.at[nidx_v], nrows_v, sn_g)

        def expand_chunk(ids_ref, rows_ref, n_groups):
            def do_group(g):
                idvec = ids_ref[pl.ds(g * _L, _L)]
                src = idvec * _E_DIM
                dstb = (g // 8) * (8 * _BLK) + (g % 8) * _L
                for j in range(_E_DIM):
                    vals = plsc.load_gather(etab_v, [src + j])
                    off = dstb + (j // 8) * _CBUF_PLANE + (j % 8) * _BLK
                    rows_ref[pl.ds(off, _L)] = vals

            def body(it, _):
                for u in range(_UNROLL):
                    do_group(it * _UNROLL + u)
                return 0
            lax.fori_loop(0, n_groups // _UNROLL, body, 0)

        wr = [(), ()]
        nw = None
        n_chunks = len(_CHUNK_SIZES)
        for c in range(n_chunks):
            b = c & 1
            tb = _CHUNK_SIZES[c]
            ecp[b].wait()
            for h in wr[b]:
                h.wait()
            expand_chunk(eidx[b], cbuf[b], tb * 8)
            if c + 2 < n_chunks:
                nxt = c + 2
                ecp[b] = pltpu.async_copy(
                    edge_ids_hbm.at[pl.ds((bstart + csum[nxt]) * _BLK,
                                          _CHUNK_SIZES[nxt] * _BLK)],
                    eidx[b].at[pl.ds(0, _CHUNK_SIZES[nxt] * _BLK)], seid[b])
            cb0 = bstart + csum[c]
            wr[b] = tuple(
                pltpu.async_copy(
                    cbuf[b].at[pl.ds(tj * _CBUF_PLANE, tb * _BLK * 8)],
                    edge_out.at[pl.ds(tj * _PLANE_STRIDE + cb0 * (_BLK * 8),
                                      tb * _BLK * 8)],
                    sew[b])
                for tj in range(2)
            )
            if c == 0:
                ng.wait()
                nw = pltpu.async_copy(
                    nrows_v, node_out.at[pl.ds(nbase, _NODE_CHUNK)], sn_w)
        for hs in wr:
            for h in hs:
                h.wait()
        nw.wait()

    return k(node_ids, edge_ids, node_table, edge_table)


def kernel(node_ids, edge_ids, node_table, edge_table):
    node_feat, edge_flat = _sc_lookup(node_ids, edge_ids, node_table,
                                      edge_table)
    edge_feat = (edge_flat.reshape(2, _N_BLKS, 8, _BLK)
                 .transpose(1, 3, 0, 2)
                 .reshape(_N_EDGES, _E_DIM))
    return node_feat, edge_feat


# speedup vs baseline: 11.8703x; 1.7063x over previous
import functools

import jax
import jax.numpy as jnp
from jax import lax
from jax.experimental import pallas as pl
from jax.experimental.pallas import tpu as pltpu
from jax.experimental.pallas import tpu_sc as plsc

_N_NODES = 10000
_N_EDGES = 320000
_NODE_DIM = 128
_E_DIM = 16
_E_VOCAB = 16

_INFO = plsc.get_sparse_core_info()
_NC, _NS = _INFO.num_cores, _INFO.num_subcores
_NW = _NC * _NS
_L = 16

_NODE_CHUNK = 320

_BLK = 128
_N_BLKS = _N_EDGES // _BLK
_NB_W = 79
_TB = 8
_CHUNK_SIZES = (8, 8, 8, 8, 8, 8, 8, 8, 8, 7)
_CBUF_PLANE = _TB * _BLK * 8
_PLANE_STRIDE = _N_BLKS * _BLK * 8
_UNROLL = 4


def _sc_lookup(node_ids, edge_ids, node_table, edge_table):
    mesh = plsc.VectorSubcoreMesh(core_axis_name="c", subcore_axis_name="s")

    @functools.partial(
        pl.kernel,
        mesh=mesh,
        out_type=(
            jax.ShapeDtypeStruct((_N_NODES, _NODE_DIM), jnp.float32),
            jax.ShapeDtypeStruct((_N_EDGES * _E_DIM,), jnp.float32),
        ),
        scratch_types=[
            pltpu.VMEM((_NODE_CHUNK,), jnp.int32),
            pltpu.VMEM((_NODE_CHUNK, _NODE_DIM), jnp.float32),
            pltpu.VMEM((_TB * _BLK,), jnp.int32),
            pltpu.VMEM((_TB * _BLK,), jnp.int32),
            pltpu.VMEM((2 * _CBUF_PLANE,), jnp.float32),
            pltpu.VMEM((2 * _CBUF_PLANE,), jnp.float32),
            pltpu.VMEM((_E_VOCAB * _E_DIM * _L,), jnp.float32),
            pltpu.SemaphoreType.DMA,
            pltpu.SemaphoreType.DMA,
            pltpu.SemaphoreType.DMA,
            pltpu.SemaphoreType.DMA,
            pltpu.SemaphoreType.DMA,
            pltpu.SemaphoreType.DMA,
            pltpu.SemaphoreType.DMA,
        ],
        compiler_params=pltpu.CompilerParams(needs_layout_passes=False),
    )
    def k(node_ids_hbm, edge_ids_hbm, node_tab_hbm, edge_tab_hbm,
          node_out, edge_out, nidx_v, nrows_v, eidx0, eidx1, cbuf0, cbuf1,
          etab_v, sn_id, sn_g, sn_w, se_id0, se_id1, se_w0, se_w1):
        wid = lax.axis_index("s") * _NC + lax.axis_index("c")
        nbase = jnp.where(wid < _NW - 1,
                          wid * _NODE_CHUNK, _N_NODES - _NODE_CHUNK)
        bstart = (wid * _N_BLKS) // _NW

        nid_cp = pltpu.async_copy(
            node_ids_hbm.at[pl.ds(nbase, _NODE_CHUNK)], nidx_v, sn_id)
        eidx = (eidx0, eidx1)
        cbuf = (cbuf0, cbuf1)
        seid = (se_id0, se_id1)
        sew = (se_w0, se_w1)
        csum = [sum(_CHUNK_SIZES[:n]) for n in range(len(_CHUNK_SIZES))]
        ecp = [
            pltpu.async_copy(
                edge_ids_hbm.at[pl.ds((bstart + csum[c]) * _BLK,
                                      _CHUNK_SIZES[c] * _BLK)],
                eidx[c].at[pl.ds(0, _CHUNK_SIZES[c] * _BLK)], seid[c])
            for c in range(2)
        ]
        pltpu.sync_copy(edge_tab_hbm, etab_v)
        nid_cp.wait()
        ng = pltpu.async_copy(node_tab_hbm.at[nidx_v], nrows_v, sn_g)

        lanes = lax.iota(jnp.int32, _L)

        def expand_chunk(ids_ref, rows_ref, n_groups):
            def do_group(g):
                idvec = ids_ref[pl.ds(g * _L, _L)]
                src = idvec * (_E_DIM * _L) + lanes
                dstb = (g // 8) * (8 * _BLK) + (g % 8) * _L
                for j in range(_E_DIM):
                    vals = plsc.load_gather(etab_v, [src + j * _L])
                    off = dstb + (j // 8) * _CBUF_PLANE + (j % 8) * _BLK
                    rows_ref[pl.ds(off, _L)] = vals

            def body(it, _):
                for u in range(_UNROLL):
                    do_group(it * _UNROLL + u)
                return 0
            lax.fori_loop(0, n_groups // _UNROLL, body, 0)

        wr = [(), ()]
        nw = None
        n_chunks = len(_CHUNK_SIZES)
        for c in range(n_chunks):
            b = c & 1
            tb = _CHUNK_SIZES[c]
            ecp[b].wait()
            for h in wr[b]:
                h.wait()
            expand_chunk(eidx[b], cbuf[b], tb * 8)
            if c + 2 < n_chunks:
                nxt = c + 2
                ecp[b] = pltpu.async_copy(
                    edge_ids_hbm.at[pl.ds((bstart + csum[nxt]) * _BLK,
                                          _CHUNK_SIZES[nxt] * _BLK)],
                    eidx[b].at[pl.ds(0, _CHUNK_SIZES[nxt] * _BLK)], seid[b])
            cb0 = bstart + csum[c]
            wr[b] = tuple(
                pltpu.async_copy(
                    cbuf[b].at[pl.ds(tj * _CBUF_PLANE, tb * _BLK * 8)],
                    edge_out.at[pl.ds(tj * _PLANE_STRIDE + cb0 * (_BLK * 8),
                                      tb * _BLK * 8)],
                    sew[b])
                for tj in range(2)
            )
            if c == 0:
                ng.wait()
                nw = pltpu.async_copy(
                    nrows_v, node_out.at[pl.ds(nbase, _NODE_CHUNK)], sn_w)
        for hs in wr:
            for h in hs:
                h.wait()
        nw.wait()

    return k(node_ids, edge_ids, node_table, edge_table)


def kernel(node_ids, edge_ids, node_table, edge_table):
    etab_rep = jnp.repeat(edge_table.reshape(-1), _L)
    node_feat, edge_flat = _sc_lookup(node_ids, edge_ids, node_table,
                                      etab_rep)
    edge_feat = (edge_flat.reshape(2, _N_BLKS, 8, _BLK)
                 .transpose(1, 3, 0, 2)
                 .reshape(_N_EDGES, _E_DIM))
    return node_feat, edge_feat


# speedup vs baseline: 12.0442x; 1.0147x over previous
import functools

import jax
import jax.numpy as jnp
from jax import lax
from jax.experimental import pallas as pl
from jax.experimental.pallas import tpu as pltpu
from jax.experimental.pallas import tpu_sc as plsc

_N_NODES = 10000
_N_EDGES = 320000
_NODE_DIM = 128
_E_DIM = 16
_E_VOCAB = 16

_INFO = plsc.get_sparse_core_info()
_NC, _NS = _INFO.num_cores, _INFO.num_subcores
_NW = _NC * _NS
_L = 16

_NODE_CHUNK = 320

_BLK = 128
_N_BLKS = _N_EDGES // _BLK
_NB_W = 79
_TB = 16
_CHUNK_SIZES = (16, 16, 16, 16, 15)
_CBUF_PLANE = _TB * _BLK * 8
_PLANE_STRIDE = _N_BLKS * _BLK * 8
_UNROLL = 8


def _sc_lookup(node_ids, edge_ids, node_table, edge_table):
    mesh = plsc.VectorSubcoreMesh(core_axis_name="c", subcore_axis_name="s")

    @functools.partial(
        pl.kernel,
        mesh=mesh,
        out_type=(
            jax.ShapeDtypeStruct((_N_NODES, _NODE_DIM), jnp.float32),
            jax.ShapeDtypeStruct((_N_EDGES * _E_DIM,), jnp.float32),
        ),
        scratch_types=[
            pltpu.VMEM((_NODE_CHUNK,), jnp.int32),
            pltpu.VMEM((_NODE_CHUNK, _NODE_DIM), jnp.float32),
            pltpu.VMEM((_TB * _BLK,), jnp.int32),
            pltpu.VMEM((_TB * _BLK,), jnp.int32),
            pltpu.VMEM((2 * _CBUF_PLANE,), jnp.float32),
            pltpu.VMEM((2 * _CBUF_PLANE,), jnp.float32),
            pltpu.VMEM((_E_VOCAB * _E_DIM * _L,), jnp.float32),
            pltpu.SemaphoreType.DMA,
            pltpu.SemaphoreType.DMA,
            pltpu.SemaphoreType.DMA,
            pltpu.SemaphoreType.DMA,
            pltpu.SemaphoreType.DMA,
            pltpu.SemaphoreType.DMA,
            pltpu.SemaphoreType.DMA,
        ],
        compiler_params=pltpu.CompilerParams(needs_layout_passes=False),
    )
    def k(node_ids_hbm, edge_ids_hbm, node_tab_hbm, edge_tab_hbm,
          node_out, edge_out, nidx_v, nrows_v, eidx0, eidx1, cbuf0, cbuf1,
          etab_v, sn_id, sn_g, sn_w, se_id0, se_id1, se_w0, se_w1):
        wid = lax.axis_index("s") * _NC + lax.axis_index("c")
        nbase = jnp.where(wid < _NW - 1,
                          wid * _NODE_CHUNK, _N_NODES - _NODE_CHUNK)
        bstart = (wid * _N_BLKS) // _NW

        nid_cp = pltpu.async_copy(
            node_ids_hbm.at[pl.ds(nbase, _NODE_CHUNK)], nidx_v, sn_id)
        eidx = (eidx0, eidx1)
        cbuf = (cbuf0, cbuf1)
        seid = (se_id0, se_id1)
        sew = (se_w0, se_w1)
        csum = [sum(_CHUNK_SIZES[:n]) for n in range(len(_CHUNK_SIZES))]
        ecp = [
            pltpu.async_copy(
                edge_ids_hbm.at[pl.ds((bstart + csum[c]) * _BLK,
                                      _CHUNK_SIZES[c] * _BLK)],
                eidx[c].at[pl.ds(0, _CHUNK_SIZES[c] * _BLK)], seid[c])
            for c in range(2)
        ]
        pltpu.sync_copy(edge_tab_hbm, etab_v)
        nid_cp.wait()
        ng = pltpu.async_copy(node_tab_hbm.at[nidx_v], nrows_v, sn_g)

        lanes = lax.iota(jnp.int32, _L)

        def expand_chunk(ids_ref, rows_ref, n_groups):
            def do_group(g):
                idvec = ids_ref[pl.ds(g * _L, _L)]
                src = idvec * (_E_DIM * _L) + lanes
                dstb = (g // 8) * (8 * _BLK) + (g % 8) * _L
                for j in range(_E_DIM):
                    vals = plsc.load_gather(etab_v, [src + j * _L])
                    off = dstb + (j // 8) * _CBUF_PLANE + (j % 8) * _BLK
                    rows_ref[pl.ds(off, _L)] = vals

            def body(it, _):
                for u in range(_UNROLL):
                    do_group(it * _UNROLL + u)
                return 0
            lax.fori_loop(0, n_groups // _UNROLL, body, 0)

        wr = [(), ()]
        nw = None
        n_chunks = len(_CHUNK_SIZES)
        for c in range(n_chunks):
            b = c & 1
            tb = _CHUNK_SIZES[c]
            ecp[b].wait()
            for h in wr[b]:
                h.wait()
            expand_chunk(eidx[b], cbuf[b], tb * 8)
            if c + 2 < n_chunks:
                nxt = c + 2
                ecp[b] = pltpu.async_copy(
                    edge_ids_hbm.at[pl.ds((bstart + csum[nxt]) * _BLK,
                                          _CHUNK_SIZES[nxt] * _BLK)],
                    eidx[b].at[pl.ds(0, _CHUNK_SIZES[nxt] * _BLK)], seid[b])
            cb0 = bstart + csum[c]
            wr[b] = tuple(
                pltpu.async_copy(
                    cbuf[b].at[pl.ds(tj * _CBUF_PLANE, tb * _BLK * 8)],
                    edge_out.at[pl.ds(tj * _PLANE_STRIDE + cb0 * (_BLK * 8),
                                      tb * _BLK * 8)],
                    sew[b])
                for tj in range(2)
            )
            if c == 1:
                ng.wait()
                nw = pltpu.async_copy(
                    nrows_v, node_out.at[pl.ds(nbase, _NODE_CHUNK)], sn_w)
        for hs in wr:
            for h in hs:
                h.wait()
        nw.wait()

    return k(node_ids, edge_ids, node_table, edge_table)


def kernel(node_ids, edge_ids, node_table, edge_table):
    etab_rep = jnp.repeat(edge_table.reshape(-1), _L)
    node_feat, edge_flat = _sc_lookup(node_ids, edge_ids, node_table,
                                      etab_rep)
    edge_feat = (edge_flat.reshape(2, _N_BLKS, 8, _BLK)
                 .transpose(1, 3, 0, 2)
                 .reshape(_N_EDGES, _E_DIM))
    return node_feat, edge_feat


# speedup vs baseline: 19.5968x; 1.6271x over previous
import functools

import jax
import jax.numpy as jnp
from jax import lax
from jax.experimental import pallas as pl
from jax.experimental.pallas import tpu as pltpu
from jax.experimental.pallas import tpu_sc as plsc

_N_NODES = 10000
_N_EDGES = 320000
_NODE_DIM = 128
_E_DIM = 16
_E_VOCAB = 16

_INFO = plsc.get_sparse_core_info()
_NC, _NS = _INFO.num_cores, _INFO.num_subcores
_NW = _NC * _NS
_L = 16

_NODE_CHUNK = 320

_BLK = 128
_N_BLKS = _N_EDGES // _BLK
_NB_W = 79
_TB = 16
_CHUNK_SIZES = (16, 16, 16, 16, 15)
_CBUF_PLANE = _TB * _BLK * 8
_PLANE_STRIDE = _N_BLKS * _BLK * 8
_UNROLL = 4


def _sc_lookup(node_ids, edge_ids, node_table, edge_table):
    mesh = plsc.VectorSubcoreMesh(core_axis_name="c", subcore_axis_name="s")

    @functools.partial(
        pl.kernel,
        mesh=mesh,
        out_type=(
            jax.ShapeDtypeStruct((_N_NODES, _NODE_DIM), jnp.float32),
            jax.ShapeDtypeStruct((_N_EDGES * _E_DIM,), jnp.float32),
        ),
        scratch_types=[
            pltpu.VMEM((_NODE_CHUNK,), jnp.int32),
            pltpu.VMEM((_NODE_CHUNK, _NODE_DIM), jnp.float32),
            pltpu.VMEM((_TB * _BLK,), jnp.int32),
            pltpu.VMEM((_TB * _BLK,), jnp.int32),
            pltpu.VMEM((2 * _CBUF_PLANE,), jnp.float32),
            pltpu.VMEM((2 * _CBUF_PLANE,), jnp.float32),
            pltpu.VMEM((_E_VOCAB * _E_DIM * _L,), jnp.float32),
            pltpu.SemaphoreType.DMA,
            pltpu.SemaphoreType.DMA,
            pltpu.SemaphoreType.DMA,
            pltpu.SemaphoreType.DMA,
            pltpu.SemaphoreType.DMA,
            pltpu.SemaphoreType.DMA,
            pltpu.SemaphoreType.DMA,
        ],
        compiler_params=pltpu.CompilerParams(needs_layout_passes=False),
    )
    def k(node_ids_hbm, edge_ids_hbm, node_tab_hbm, edge_tab_hbm,
          node_out, edge_out, nidx_v, nrows_v, eidx0, eidx1, cbuf0, cbuf1,
          etab_v, sn_id, sn_g, sn_w, se_id0, se_id1, se_w0, se_w1):
        wid = lax.axis_index("s") * _NC + lax.axis_index("c")
        nbase = jnp.where(wid < _NW - 1,
                          wid * _NODE_CHUNK, _N_NODES - _NODE_CHUNK)
        bstart = (wid * _N_BLKS) // _NW

        nid_cp = pltpu.async_copy(
            node_ids_hbm.at[pl.ds(nbase, _NODE_CHUNK)], nidx_v, sn_id)
        eidx = (eidx0, eidx1)
        cbuf = (cbuf0, cbuf1)
        seid = (se_id0, se_id1)
        sew = (se_w0, se_w1)
        csum = [sum(_CHUNK_SIZES[:n]) for n in range(len(_CHUNK_SIZES))]
        ecp = [
            pltpu.async_copy(
                edge_ids_hbm.at[pl.ds((bstart + csum[c]) * _BLK,
                                      _CHUNK_SIZES[c] * _BLK)],
                eidx[c].at[pl.ds(0, _CHUNK_SIZES[c] * _BLK)], seid[c])
            for c in range(2)
        ]
        pltpu.sync_copy(edge_tab_hbm, etab_v)
        nid_cp.wait()
        ng = pltpu.async_copy(node_tab_hbm.at[nidx_v], nrows_v, sn_g)

        lanes = lax.iota(jnp.int32, _L)

        def expand_chunk(ids_ref, rows_ref, n_groups):
            def do_group(g):
                idvec = ids_ref[pl.ds(g * _L, _L)]
                src = idvec * (_E_DIM * _L) + lanes
                dstb = (g // 8) * (8 * _BLK) + (g % 8) * _L
                vals = [plsc.load_gather(etab_v, [src + j * _L])
                        for j in range(_E_DIM)]
                for j in range(_E_DIM):
                    off = dstb + (j // 8) * _CBUF_PLANE + (j % 8) * _BLK
                    rows_ref[pl.ds(off, _L)] = vals[j]

            def body(it, _):
                for u in range(_UNROLL):
                    do_group(it * _UNROLL + u)
                return 0
            lax.fori_loop(0, n_groups // _UNROLL, body, 0)

        wr = [(), ()]
        nw = None
        n_chunks = len(_CHUNK_SIZES)
        for c in range(n_chunks):
            b = c & 1
            tb = _CHUNK_SIZES[c]
            ecp[b].wait()
            for h in wr[b]:
                h.wait()
            expand_chunk(eidx[b], cbuf[b], tb * 8)
            if c + 2 < n_chunks:
                nxt = c + 2
                ecp[b] = pltpu.async_copy(
                    edge_ids_hbm.at[pl.ds((bstart + csum[nxt]) * _BLK,
                                          _CHUNK_SIZES[nxt] * _BLK)],
                    eidx[b].at[pl.ds(0, _CHUNK_SIZES[nxt] * _BLK)], seid[b])
            cb0 = bstart + csum[c]
            wr[b] = tuple(
                pltpu.async_copy(
                    cbuf[b].at[pl.ds(tj * _CBUF_PLANE, tb * _BLK * 8)],
                    edge_out.at[pl.ds(tj * _PLANE_STRIDE + cb0 * (_BLK * 8),
                                      tb * _BLK * 8)],
                    sew[b])
                for tj in range(2)
            )
            if c == 1:
                ng.wait()
                nw = pltpu.async_copy(
                    nrows_v, node_out.at[pl.ds(nbase, _NODE_CHUNK)], sn_w)
        for hs in wr:
            for h in hs:
                h.wait()
        nw.wait()

    return k(node_ids, edge_ids, node_table, edge_table)


def kernel(node_ids, edge_ids, node_table, edge_table):
    etab_rep = jnp.repeat(edge_table.reshape(-1), _L)
    node_feat, edge_flat = _sc_lookup(node_ids, edge_ids, node_table,
                                      etab_rep)
    edge_feat = (edge_flat.reshape(2, _N_BLKS, 8, _BLK)
                 .transpose(1, 3, 0, 2)
                 .reshape(_N_EDGES, _E_DIM))
    return node_feat, edge_feat


# speedup vs baseline: 20.2061x; 1.0311x over previous
import functools

import jax
import jax.numpy as jnp
from jax import lax
from jax.experimental import pallas as pl
from jax.experimental.pallas import tpu as pltpu
from jax.experimental.pallas import tpu_sc as plsc

_N_NODES = 10000
_N_EDGES = 320000
_NODE_DIM = 128
_E_DIM = 16
_E_VOCAB = 16

_INFO = plsc.get_sparse_core_info()
_NC, _NS = _INFO.num_cores, _INFO.num_subcores
_NW = _NC * _NS
_L = 16

_NODE_CHUNK = 320

_BLK = 128
_N_BLKS = _N_EDGES // _BLK
_NB_W = 79
_TB = 16
_CHUNK_SIZES = (16, 16, 16, 16, 15)
_CBUF_PLANE = _TB * _BLK * 8
_PLANE_STRIDE = _N_BLKS * _BLK * 8
_UNROLL = 4


def _sc_lookup(node_ids, edge_ids, node_table, edge_table):
    mesh = plsc.VectorSubcoreMesh(core_axis_name="c", subcore_axis_name="s")

    @functools.partial(
        pl.kernel,
        mesh=mesh,
        out_type=(
            jax.ShapeDtypeStruct((_N_NODES, _NODE_DIM), jnp.float32),
            jax.ShapeDtypeStruct((_N_EDGES * _E_DIM,), jnp.float32),
        ),
        scratch_types=[
            pltpu.VMEM((_NODE_CHUNK,), jnp.int32),
            pltpu.VMEM((_NODE_CHUNK, _NODE_DIM), jnp.float32),
            pltpu.VMEM((_TB * _BLK,), jnp.int32),
            pltpu.VMEM((_TB * _BLK,), jnp.int32),
            pltpu.VMEM((2 * _CBUF_PLANE,), jnp.float32),
            pltpu.VMEM((2 * _CBUF_PLANE,), jnp.float32),
            pltpu.VMEM((_E_VOCAB * _E_DIM * _L,), jnp.float32),
            pltpu.SemaphoreType.DMA,
            pltpu.SemaphoreType.DMA,
            pltpu.SemaphoreType.DMA,
            pltpu.SemaphoreType.DMA,
            pltpu.SemaphoreType.DMA,
            pltpu.SemaphoreType.DMA,
            pltpu.SemaphoreType.DMA,
        ],
        compiler_params=pltpu.CompilerParams(needs_layout_passes=False),
    )
    def k(node_ids_hbm, edge_ids_hbm, node_tab_hbm, edge_tab_hbm,
          node_out, edge_out, nidx_v, nrows_v, eidx0, eidx1, cbuf0, cbuf1,
          etab_v, sn_id, sn_g, sn_w, se_id0, se_id1, se_w0, se_w1):
        wid = lax.axis_index("s") * _NC + lax.axis_index("c")
        nbase = jnp.where(wid < _NW - 1,
                          wid * _NODE_CHUNK, _N_NODES - _NODE_CHUNK)
        bstart = (wid * _N_BLKS) // _NW

        nid_cp = pltpu.async_copy(
            node_ids_hbm.at[pl.ds(nbase, _NODE_CHUNK)], nidx_v, sn_id)
        eidx = (eidx0, eidx1)
        cbuf = (cbuf0, cbuf1)
        seid = (se_id0, se_id1)
        sew = (se_w0, se_w1)
        csum = [sum(_CHUNK_SIZES[:n]) for n in range(len(_CHUNK_SIZES))]
        ecp = [
            pltpu.async_copy(
                edge_ids_hbm.at[pl.ds((bstart + csum[c]) * _BLK,
                                      _CHUNK_SIZES[c] * _BLK)],
                eidx[c].at[pl.ds(0, _CHUNK_SIZES[c] * _BLK)], seid[c])
            for c in range(2)
        ]
        pltpu.sync_copy(edge_tab_hbm, etab_v)
        nid_cp.wait()
        ng = pltpu.async_copy(node_tab_hbm.at[nidx_v], nrows_v, sn_g)

        lanes = lax.iota(jnp.int32, _L)

        def expand_chunk(ids_ref, rows_ref, n_groups):
            def gather_group(g):
                idvec = ids_ref[pl.ds(g * _L, _L)]
                src = idvec * (_E_DIM * _L) + lanes
                return tuple(plsc.load_gather(etab_v, [src + j * _L])
                             for j in range(_E_DIM))

            def store_group(g, vals):
                dstb = (g // 8) * (8 * _BLK) + (g % 8) * _L
                for j in range(_E_DIM):
                    off = dstb + (j // 8) * _CBUF_PLANE + (j % 8) * _BLK
                    rows_ref[pl.ds(off, _L)] = vals[j]

            def body(it, _):
                g0 = it * _UNROLL
                prev = gather_group(g0)
                for u in range(1, _UNROLL):
                    cur = gather_group(g0 + u)
                    store_group(g0 + u - 1, prev)
                    prev = cur
                store_group(g0 + _UNROLL - 1, prev)
                return 0

            lax.fori_loop(0, n_groups // _UNROLL, body, 0)

        wr = [(), ()]
        nw = None
        n_chunks = len(_CHUNK_SIZES)
        for c in range(n_chunks):
            b = c & 1
            tb = _CHUNK_SIZES[c]
            ecp[b].wait()
            for h in wr[b]:
                h.wait()
            expand_chunk(eidx[b], cbuf[b], tb * 8)
            if c + 2 < n_chunks:
                nxt = c + 2
                ecp[b] = pltpu.async_copy(
                    edge_ids_hbm.at[pl.ds((bstart + csum[nxt]) * _BLK,
                                          _CHUNK_SIZES[nxt] * _BLK)],
                    eidx[b].at[pl.ds(0, _CHUNK_SIZES[nxt] * _BLK)], seid[b])
            cb0 = bstart + csum[c]
            wr[b] = tuple(
                pltpu.async_copy(
                    cbuf[b].at[pl.ds(tj * _CBUF_PLANE, tb * _BLK * 8)],
                    edge_out.at[pl.ds(tj * _PLANE_STRIDE + cb0 * (_BLK * 8),
                                      tb * _BLK * 8)],
                    sew[b])
                for tj in range(2)
            )
            if c == 1:
                ng.wait()
                nw = pltpu.async_copy(
                    nrows_v, node_out.at[pl.ds(nbase, _NODE_CHUNK)], sn_w)
        for hs in wr:
            for h in hs:
                h.wait()
        nw.wait()

    return k(node_ids, edge_ids, node_table, edge_table)


def kernel(node_ids, edge_ids, node_table, edge_table):
    etab_rep = jnp.repeat(edge_table.reshape(-1), _L)
    node_feat, edge_flat = _sc_lookup(node_ids, edge_ids, node_table,
                                      etab_rep)
    edge_feat = (edge_flat.reshape(2, _N_BLKS, 8, _BLK)
                 .transpose(1, 3, 0, 2)
                 .reshape(_N_EDGES, _E_DIM))
    return node_feat, edge_feat
